# baseline (speedup 1.0000x reference)
import functools

import jax
import jax.numpy as jnp
from jax import lax
from jax.experimental import pallas as pl
from jax.experimental.pallas import tpu as pltpu
from jax.experimental.pallas import tpu_sc as plsc

BLK = 128
PR = 128
NG = 2


@functools.lru_cache(maxsize=None)
def _make_gather(s: int, b: int, d: int):
    info = plsc.get_sparse_core_info()
    nc = info.num_cores
    nw = nc * info.num_subcores
    assert b == nw * BLK and d % 8 == 0 and s % 4 == 0 and s >= 12
    mesh = plsc.VectorSubcoreMesh(core_axis_name="c", subcore_axis_name="s")

    @functools.partial(
        pl.kernel,
        mesh=mesh,
        out_type=jax.ShapeDtypeStruct((s, d // 8, nw, 8, BLK), jnp.float32),
        compiler_params=pltpu.CompilerParams(
            use_tc_tiling_on_sc=True, needs_layout_passes=False
        ),
        scratch_types=[
            pltpu.VMEM((s, BLK), jnp.int32),
            pltpu.VMEM((NG, 1, BLK), jnp.int32),
            pltpu.VMEM((NG, BLK, PR), jnp.float32),
            pltpu.VMEM((2, d // 8, 8, BLK), jnp.float32),
            pltpu.SemaphoreType.DMA((NG,)),
            pltpu.SemaphoreType.DMA((2,)),
        ],
    )
    def gather_kernel(
        idx_hbm, table_hbm, out_hbm, idx_v, row_idx_v, rows_v, rowst_v, gsem, osem
    ):
        wid = lax.axis_index("s") * nc + lax.axis_index("c")
        col0 = wid * BLK
        pltpu.sync_copy(idx_hbm.at[:, pl.ds(col0, BLK)], idx_v)

        iota = lax.iota(jnp.int32, 16)

        def fire_gather(l, a):
            for g in range(BLK // 16):
                sl = pl.ds(g * 16, 16)
                row_idx_v[a, 0, sl] = idx_v[l, sl] >> 1
            pltpu.async_copy(
                table_hbm.at[row_idx_v.at[a, 0]], rows_v.at[a], gsem.at[a]
            )

        def wait_gather(a):
            pltpu.make_async_copy(
                table_hbm.at[row_idx_v.at[a, 0]], rows_v.at[a], gsem.at[a]
            ).wait()

        def out_pair(l, a):
            return rowst_v.at[a], out_hbm.at[l, :, wid, :, :]

        def fire_out(l, a):
            src, dst = out_pair(l, a)
            pltpu.async_copy(src, dst, osem.at[a])

        def wait_out(l, a):
            src, dst = out_pair(l, a)
            pltpu.make_async_copy(src, dst, osem.at[a]).wait()

        def transpose(l, a, bo):
            src = rows_v.at[a]
            dst = rowst_v.at[bo]
            for g in range(BLK // 16):
                c = g * 16 + iota
                sl = pl.ds(g * 16, 16)
                poff = (idx_v[l, sl] & 1) << 6
                for f0 in range(0, d, 8):
                    vs = [
                        plsc.load_gather(src, [c, poff + (f0 + u)])
                        for u in range(8)
                    ]
                    for u in range(8):
                        f = f0 + u
                        dst[f // 8, f % 8, sl] = vs[u]

        def step(l, a, bo, fire, first):
            if fire:
                fire_gather(l + (NG - 1), (a + NG - 1) % NG)
            wait_gather(a)
            if not first:
                wait_out(l - 2, bo)
            transpose(l, a, bo)
            fire_out(l, bo)

        for j in range(NG - 1):
            fire_gather(j, j % NG)
        step(0, 0, 0, True, True)
        step(1, 1 % NG, 1, True, True)

        def group(gi, _):
            l = 2 * gi
            step(l, 0, 0, True, False)
            step(l + 1, 1 % NG, 1, True, False)
            return ()

        lax.fori_loop(1, s // 2 - 1, group, ())

        step(s - 2, 0, 0, True, False)
        step(s - 1, 1 % NG, 1, False, False)
        wait_out(s - 2, 0)
        wait_out(s - 1, 1)

    return gather_kernel


def kernel(token_ids, table):
    b, s = token_ids.shape
    d = table.shape[1]
    idx_t = token_ids.T.astype(jnp.int32)
    table2 = table.reshape(-1, PR)
    out5 = _make_gather(s, b, d)(idx_t, table2)
    return out5.transpose(2, 4, 0, 1, 3).reshape(b, s, d)


# speedup vs baseline: 1.0807x; 1.0807x over previous
import functools

import jax
import jax.numpy as jnp
from jax import lax
from jax.experimental import pallas as pl
from jax.experimental.pallas import tpu as pltpu
from jax.experimental.pallas import tpu_sc as plsc

BLK = 128
PADD = 128
NG = 4


@functools.lru_cache(maxsize=None)
def _make_gather(s: int, b: int, d: int):
    info = plsc.get_sparse_core_info()
    nc = info.num_cores
    nw = nc * info.num_subcores
    assert b == nw * BLK and d % 8 == 0 and s % 4 == 0 and s >= 12
    mesh = plsc.VectorSubcoreMesh(core_axis_name="c", subcore_axis_name="s")

    @functools.partial(
        pl.kernel,
        mesh=mesh,
        out_type=jax.ShapeDtypeStruct((s, d // 8, nw, 8, BLK), jnp.float32),
        compiler_params=pltpu.CompilerParams(
            use_tc_tiling_on_sc=True, needs_layout_passes=False
        ),
        scratch_types=[
            pltpu.VMEM((s, BLK), jnp.int32),
            pltpu.VMEM((NG, BLK, PADD), jnp.float32),
            pltpu.VMEM((2, d // 8, 8, BLK), jnp.float32),
            pltpu.SemaphoreType.DMA((NG,)),
            pltpu.SemaphoreType.DMA((2,)),
        ],
    )
    def gather_kernel(idx_hbm, table_hbm, out_hbm, idx_v, rows_v, rowst_v, gsem, osem):
        wid = lax.axis_index("s") * nc + lax.axis_index("c")
        col0 = wid * BLK
        pltpu.sync_copy(idx_hbm.at[:, pl.ds(col0, BLK)], idx_v)

        iota = lax.iota(jnp.int32, 16)
        fsplats = [jnp.full((16,), f, jnp.int32) for f in range(d)]

        def fire_gather(l, a):
            pltpu.async_copy(
                table_hbm.at[idx_v.at[l]], rows_v.at[a], gsem.at[a]
            )

        def wait_gather(l, a):
            pltpu.make_async_copy(
                table_hbm.at[idx_v.at[l]], rows_v.at[a], gsem.at[a]
            ).wait()

        def out_pair(l, bo):
            return rowst_v.at[bo], out_hbm.at[l, :, wid, :, :]

        def fire_out(l, bo):
            src, dst = out_pair(l, bo)
            pltpu.async_copy(src, dst, osem.at[bo])

        def wait_out(l, bo):
            src, dst = out_pair(l, bo)
            pltpu.make_async_copy(src, dst, osem.at[bo]).wait()

        def transpose(a, bo):
            src = rows_v.at[a]
            dst = rowst_v.at[bo]

            def gbody(g, _):
                c = g * 16 + iota
                for f0 in range(0, d, 8):
                    vs = [
                        plsc.load_gather(src, [c, fsplats[f0 + u]])
                        for u in range(8)
                    ]
                    for u in range(8):
                        f = f0 + u
                        plsc.store_scatter(dst.at[f // 8, f % 8], [c], vs[u])
                return ()

            lax.fori_loop(0, BLK // 16, gbody, ())

        def step(l, a, bo, fire, first):
            if fire:
                fire_gather(l + (NG - 1), (a + NG - 1) % NG)
            wait_gather(l, a)
            if not first:
                wait_out(l - 2, bo)
            transpose(a, bo)
            fire_out(l, bo)

        for j in range(NG - 1):
            fire_gather(j, j)
        step(0, 0, 0, True, True)
        step(1, 1, 1, True, True)
        step(2, 2, 0, True, False)
        step(3, 3, 1, True, False)

        def group(gi, _):
            l0 = 4 * gi
            for j in range(4):
                step(l0 + j, j, j % 2, True, False)
            return ()

        lax.fori_loop(1, s // 4 - 1, group, ())

        l0 = s - 4
        for j in range(4):
            step(l0 + j, j, j % 2, j == 0, False)
        wait_out(s - 2, 0)
        wait_out(s - 1, 1)

    return gather_kernel


def kernel(token_ids, table):
    b, s = token_ids.shape
    d = table.shape[1]
    idx_t = token_ids.T.astype(jnp.int32)
    table_p = jnp.pad(table, ((0, 0), (0, PADD - d)))
    out5 = _make_gather(s, b, d)(idx_t, table_p)
    return out5.transpose(2, 4, 0, 1, 3).reshape(b, s, d)


# speedup vs baseline: 1.1105x; 1.0275x over previous
import functools

import jax
import jax.numpy as jnp
from jax import lax
from jax.experimental import pallas as pl
from jax.experimental.pallas import tpu as pltpu
from jax.experimental.pallas import tpu_sc as plsc

CHUNK = 128
NBUF = 8


@functools.lru_cache(maxsize=None)
def _make_gather(n: int, d: int):
    info = plsc.get_sparse_core_info()
    nw = info.num_cores * info.num_subcores
    assert n % (nw * CHUNK) == 0
    b_per_w = n // nw
    n_chunks = b_per_w // CHUNK
    mesh = plsc.VectorSubcoreMesh(core_axis_name="c", subcore_axis_name="s")

    @functools.partial(
        pl.kernel,
        mesh=mesh,
        out_type=jax.ShapeDtypeStruct((n, d), jnp.float32),
        compiler_params=pltpu.CompilerParams(use_tc_tiling_on_sc=False),
        scratch_types=[
            pltpu.VMEM((n_chunks, CHUNK), jnp.int32),
            pltpu.VMEM((NBUF, CHUNK, d), jnp.float32),
            pltpu.SemaphoreType.DMA((NBUF,)),
            pltpu.SemaphoreType.DMA((NBUF,)),
        ],
    )
    def gather_kernel(idx_hbm, table_hbm, out_hbm, idx_v, rows_v, gsem, osem):
        wid = lax.axis_index("s") * info.num_cores + lax.axis_index("c")
        base = wid * b_per_w
        pltpu.sync_copy(idx_hbm.at[wid], idx_v)

        def fire_gather(j, b):
            pltpu.async_copy(table_hbm.at[idx_v.at[j]], rows_v.at[b], gsem.at[b])

        def wait_gather(j, b):
            pltpu.make_async_copy(
                table_hbm.at[idx_v.at[j]], rows_v.at[b], gsem.at[b]
            ).wait()

        def out_ref(j, b):
            return (rows_v.at[b], out_hbm.at[pl.ds(base + j * CHUNK, CHUNK)])

        def fire_out(j, b):
            src, dst = out_ref(j, b)
            pltpu.async_copy(src, dst, osem.at[b])

        def wait_out(j, b):
            src, dst = out_ref(j, b)
            pltpu.make_async_copy(src, dst, osem.at[b]).wait()

        n_groups = n_chunks // NBUF
        for b in range(NBUF):
            fire_gather(b, b)

        def group(g, _):
            j0 = g * NBUF
            for b in range(NBUF):
                wait_gather(j0 + b, b)
                fire_out(j0 + b, b)
            jn0 = j0 + NBUF
            for b in range(NBUF):
                wait_out(j0 + b, b)
                fire_gather(jn0 + b, b)
            return ()

        lax.fori_loop(0, n_groups - 1, group, ())

        j0 = (n_groups - 1) * NBUF
        for b in range(NBUF):
            wait_gather(j0 + b, b)
            fire_out(j0 + b, b)
        for b in range(NBUF):
            wait_out(j0 + b, b)

    return gather_kernel


def kernel(token_ids, table):
    b, s = token_ids.shape
    n = b * s
    info = plsc.get_sparse_core_info()
    nw = info.num_cores * info.num_subcores
    idx = token_ids.reshape(nw, -1, CHUNK).astype(jnp.int32)
    out = _make_gather(n, table.shape[1])(idx, table)
    return out.reshape(b, s, table.shape[1])
